# baseline (speedup 1.0000x reference)
import jax
import jax.numpy as jnp
from jax import lax
from jax.experimental import pallas as pl
from jax.experimental.pallas import tpu as pltpu
from jax.experimental.pallas import tpu_sc as plsc

N_NODES = 10000
N_EDGES = 320000
NODE_DIM = 8
EDGE_DIM = 4
HIDDEN_DIM = 32
EMBED_DIM = 64
N_GRAPHS = 64
VOCAB = 118

NW = 32
WIN = 128
KW = 80
EPW = WIN * KW
E_PAD = NW * EPW
N_PAD = 10240
D_EXT = 48

def _vec_mesh():
    return plsc.VectorSubcoreMesh(core_axis_name="c", subcore_axis_name="s",
                                  num_cores=2, num_subcores=16)


def _gather_body(xp_hbm, srcm_hbm, xs_hbm, idx_v, rows_v, sem):
    wid = lax.axis_index("c") * 16 + lax.axis_index("s")
    pltpu.sync_copy(srcm_hbm.at[pl.ds(wid * KW, KW)], idx_v)

    @pl.loop(0, KW)
    def _(j):
        pltpu.async_copy(xp_hbm.at[idx_v.at[j]], rows_v, sem).wait()
        pltpu.sync_copy(rows_v, xs_hbm.at[pl.ds(wid * EPW + j * WIN, WIN)])


def _sc_gather(xp, srcm):
    return pl.kernel(
        _gather_body,
        out_type=jax.ShapeDtypeStruct((E_PAD, 16), jnp.float32),
        mesh=_vec_mesh(),
        scratch_types=[
            pltpu.VMEM((KW, WIN), jnp.int32),
            pltpu.VMEM((WIN, 16), jnp.float32),
            pltpu.SemaphoreType.DMA,
        ],
        compiler_params=pltpu.CompilerParams(use_tc_tiling_on_sc=False),
    )(xp, srcm)


def _scatter_body(msg_hbm, dstm_hbm, out_hbm, idx_v, upd_v, zb_v, acc_sh, sem):
    c = lax.axis_index("c")
    s = lax.axis_index("s")
    wid = c * 16 + s
    stripe = N_PAD // 16

    @pl.loop(0, 16)
    def _(r):
        @pl.loop(0, D_EXT // 16)
        def _(cc):
            zb_v[r, pl.ds(cc * 16, 16)] = jnp.zeros((16,), jnp.float32)

    @pl.loop(0, stripe // 16)
    def _(t):
        pltpu.sync_copy(zb_v, acc_sh.at[pl.ds(s * stripe + t * 16, 16)])

    plsc.subcore_barrier()

    @pl.loop(0, KW)
    def _(j):
        pltpu.sync_copy(dstm_hbm.at[wid * KW + j], idx_v)
        pltpu.sync_copy(msg_hbm.at[pl.ds(wid * EPW + j * WIN, WIN)], upd_v)
        pltpu.sync_copy(upd_v, acc_sh.at[idx_v], add=True)

    plsc.subcore_barrier()
    pltpu.sync_copy(acc_sh.at[pl.ds(s * stripe, stripe)],
                    out_hbm.at[c, pl.ds(s * stripe, stripe)])


def _sc_scatter(msg, dstm):
    return pl.kernel(
        _scatter_body,
        out_type=jax.ShapeDtypeStruct((2, N_PAD, D_EXT), jnp.float32),
        mesh=_vec_mesh(),
        scratch_types=[
            pltpu.VMEM((WIN,), jnp.int32),
            pltpu.VMEM((WIN, D_EXT), jnp.float32),
            pltpu.VMEM((16, D_EXT), jnp.float32),
            pltpu.VMEM_SHARED((N_PAD, D_EXT), jnp.float32),
            pltpu.SemaphoreType.DMA,
        ],
        compiler_params=pltpu.CompilerParams(use_tc_tiling_on_sc=False),
    )(msg, dstm)


_BE = 2048


def _msg_body(ea_ref, xs_ref, W1_ref, b1_ref, W2t_ref, b2r_ref, out_ref):
    h = jnp.maximum(
        jnp.dot(ea_ref[...], W1_ref[...], preferred_element_type=jnp.float32)
        + b1_ref[...], 0.0)
    xs = xs_ref[...]
    p2 = jnp.concatenate([h * xs[:, i:i + 1] for i in range(NODE_DIM)], axis=1)
    acc = (jnp.dot(p2, W2t_ref[...], preferred_element_type=jnp.float32)
           + jnp.dot(xs[:, :NODE_DIM], b2r_ref[...],
                     preferred_element_type=jnp.float32))
    out_ref[...] = jnp.concatenate(
        [acc,
         jnp.ones((_BE, 1), jnp.float32),
         jnp.zeros((_BE, D_EXT - HIDDEN_DIM - 1), jnp.float32)], axis=1)


def _tc_messages(ea_p, xs, W1, b1r, W2t, b2r):
    grid = E_PAD // _BE
    return pl.pallas_call(
        _msg_body,
        grid=(grid,),
        in_specs=[
            pl.BlockSpec((_BE, EDGE_DIM), lambda i: (i, 0)),
            pl.BlockSpec((_BE, 16), lambda i: (i, 0)),
            pl.BlockSpec((EDGE_DIM, 32), lambda i: (0, 0)),
            pl.BlockSpec((1, 32), lambda i: (0, 0)),
            pl.BlockSpec((NODE_DIM * 32, HIDDEN_DIM), lambda i: (0, 0)),
            pl.BlockSpec((NODE_DIM, HIDDEN_DIM), lambda i: (0, 0)),
        ],
        out_specs=pl.BlockSpec((_BE, D_EXT), lambda i: (i, 0)),
        out_shape=jax.ShapeDtypeStruct((E_PAD, D_EXT), jnp.float32),
        compiler_params=pltpu.CompilerParams(
            dimension_semantics=("parallel",)),
    )(ea_p, xs, W1, b1r, W2t, b2r)


_TN = 2000
_NSTEPS = N_NODES // _TN


def _finish_body(acc_ref, x_ref, batch_ref, Wr_ref, bc_ref, ids_ref, rat_ref,
                 emb_ref, W0a_ref, W0b_ref, b0_ref, Wf1_ref, bf1_ref,
                 Wf2_ref, bf2_ref, out_ref, psum_ref, pcnt_ref):
    i = pl.program_id(0)

    @pl.when(i == 0)
    def _():
        psum_ref[...] = jnp.zeros_like(psum_ref)
        pcnt_ref[...] = jnp.zeros_like(pcnt_ref)

    a0 = acc_ref[0]
    a1 = acc_ref[1]
    msum = a0[:, :HIDDEN_DIM] + a1[:, :HIDDEN_DIM]
    cnt = a0[:, HIDDEN_DIM:HIDDEN_DIM + 1] + a1[:, HIDDEN_DIM:HIDDEN_DIM + 1]
    agg = msum / jnp.maximum(cnt, 1.0)
    h1 = jnp.maximum(
        jnp.dot(x_ref[...], Wr_ref[...], preferred_element_type=jnp.float32)
        + agg + bc_ref[...], 0.0)
    oh = (lax.broadcasted_iota(jnp.int32, (N_GRAPHS, _TN), 0)
          == batch_ref[0]).astype(jnp.float32)
    psum_ref[...] += jnp.dot(oh, h1, preferred_element_type=jnp.float32)
    pcnt_ref[...] += jnp.dot(oh, jnp.ones((_TN, HIDDEN_DIM), jnp.float32),
                             preferred_element_type=jnp.float32)

    @pl.when(i == _NSTEPS - 1)
    def _():
        pooled = psum_ref[...] / jnp.maximum(pcnt_ref[...], 1.0)
        sel = (ids_ref[...] == lax.broadcasted_iota(jnp.int32, (8, VOCAB), 1)
               ).astype(jnp.float32) * rat_ref[...]
        u = jnp.sum(jnp.dot(sel, emb_ref[...],
                            preferred_element_type=jnp.float32),
                    axis=0, keepdims=True)
        z1 = jnp.maximum(
            jnp.dot(pooled, W0a_ref[...], preferred_element_type=jnp.float32)
            + jnp.dot(u, W0b_ref[...], preferred_element_type=jnp.float32)
            + b0_ref[...], 0.0)
        z2 = jnp.maximum(
            jnp.dot(z1, Wf1_ref[...], preferred_element_type=jnp.float32)
            + bf1_ref[...], 0.0)
        out_ref[...] = (jnp.dot(z2, Wf2_ref[...],
                                preferred_element_type=jnp.float32)
                        + bf2_ref[...])


def _tc_finish(accs, x, batch_row, W_root, bcr, ids_col, rat_col, emb_table,
               W0a, W0b, b0r, Wf1, bf1r, Wf2, bf2r):
    full = lambda a: pl.BlockSpec(a.shape, lambda i: tuple(0 for _ in a.shape))
    return pl.pallas_call(
        _finish_body,
        grid=(_NSTEPS,),
        in_specs=[
            pl.BlockSpec((2, _TN, D_EXT), lambda i: (0, i, 0)),
            pl.BlockSpec((_TN, NODE_DIM), lambda i: (i, 0)),
            pl.BlockSpec((1, 1, _TN), lambda i: (i, 0, 0)),
            full(W_root), full(bcr), full(ids_col), full(rat_col),
            full(emb_table), full(W0a), full(W0b), full(b0r),
            full(Wf1), full(bf1r), full(Wf2), full(bf2r),
        ],
        out_specs=pl.BlockSpec((N_GRAPHS, 1), lambda i: (0, 0)),
        out_shape=jax.ShapeDtypeStruct((N_GRAPHS, 1), jnp.float32),
        scratch_shapes=[
            pltpu.VMEM((N_GRAPHS, HIDDEN_DIM), jnp.float32),
            pltpu.VMEM((N_GRAPHS, HIDDEN_DIM), jnp.float32),
        ],
        compiler_params=pltpu.CompilerParams(
            dimension_semantics=("arbitrary",)),
    )(accs, x, batch_row, W_root, bcr, ids_col, rat_col, emb_table,
      W0a, W0b, b0r, Wf1, bf1r, Wf2, bf2r)


def kernel(x, edge_index, edge_attr, batch, the_ids, the_ratios,
           W1, b1, W2, b2, W_root, b_conv, emb_table, W0, b0,
           Wf1, bf1, Wf2, bf2):
    pad = E_PAD - N_EDGES
    src = edge_index[0]
    dst = edge_index[1]
    srcm = jnp.concatenate(
        [src, jnp.zeros((pad,), jnp.int32)]).reshape(E_PAD // WIN, WIN)
    dstm = jnp.concatenate(
        [dst, jnp.full((pad,), N_NODES, jnp.int32)]).reshape(E_PAD // WIN, WIN)
    ea_p = jnp.concatenate(
        [edge_attr, jnp.zeros((pad, EDGE_DIM), jnp.float32)], axis=0)
    xp = jnp.pad(x, ((0, 0), (0, 16 - NODE_DIM)))

    xs = _sc_gather(xp, srcm)
    W2t = W2.reshape(32, NODE_DIM, HIDDEN_DIM).transpose(1, 0, 2).reshape(
        NODE_DIM * 32, HIDDEN_DIM)
    msg = _tc_messages(ea_p, xs, W1, b1.reshape(1, 32), W2t,
                       b2.reshape(NODE_DIM, HIDDEN_DIM))
    accs = _sc_scatter(msg, dstm)
    out = _tc_finish(accs, x, batch.reshape(_NSTEPS, 1, _TN), W_root,
                     b_conv.reshape(1, HIDDEN_DIM),
                     the_ids.reshape(8, 1), the_ratios.reshape(8, 1),
                     emb_table, W0[:HIDDEN_DIM], W0[HIDDEN_DIM:],
                     b0.reshape(1, 64), Wf1, bf1.reshape(1, 32),
                     Wf2, bf2.reshape(1, 1))
    return out


# speedup vs baseline: 1.3765x; 1.3765x over previous
import jax
import jax.numpy as jnp
from jax import lax
from jax.experimental import pallas as pl
from jax.experimental.pallas import tpu as pltpu
from jax.experimental.pallas import tpu_sc as plsc

N_NODES = 10000
N_EDGES = 320000
NODE_DIM = 8
EDGE_DIM = 4
HIDDEN_DIM = 32
EMBED_DIM = 64
N_GRAPHS = 64
VOCAB = 118

NW = 32
WIN = 128
KW = 80
EPW = WIN * KW
E_PAD = NW * EPW
N_PAD = 10240
D_EXT = 48

def _vec_mesh():
    return plsc.VectorSubcoreMesh(core_axis_name="c", subcore_axis_name="s",
                                  num_cores=2, num_subcores=16)


def _gather_body(xp_hbm, srcm_hbm, xs_hbm, idx_v, rows_v, sem):
    wid = lax.axis_index("c") * 16 + lax.axis_index("s")
    pltpu.sync_copy(srcm_hbm.at[pl.ds(wid * KW, KW)], idx_v)

    @pl.loop(0, KW)
    def _(j):
        pltpu.async_copy(xp_hbm.at[idx_v.at[j]], rows_v, sem).wait()
        pltpu.sync_copy(rows_v, xs_hbm.at[pl.ds(wid * EPW + j * WIN, WIN)])


def _sc_gather(xp, srcm):
    return pl.kernel(
        _gather_body,
        out_type=jax.ShapeDtypeStruct((E_PAD, 16), jnp.float32),
        mesh=_vec_mesh(),
        scratch_types=[
            pltpu.VMEM((KW, WIN), jnp.int32),
            pltpu.VMEM((WIN, 16), jnp.float32),
            pltpu.SemaphoreType.DMA,
        ],
        compiler_params=pltpu.CompilerParams(use_tc_tiling_on_sc=False),
    )(xp, srcm)


def _scatter_body(msg_hbm, dstm_hbm, out_hbm, idx_v, upd_v, zb_v, acc_sh, sem):
    c = lax.axis_index("c")
    s = lax.axis_index("s")
    wid = c * 16 + s
    stripe = N_PAD // 16

    @pl.loop(0, 16)
    def _(r):
        @pl.loop(0, D_EXT // 16)
        def _(cc):
            zb_v[r, pl.ds(cc * 16, 16)] = jnp.zeros((16,), jnp.float32)

    @pl.loop(0, stripe // 16)
    def _(t):
        pltpu.sync_copy(zb_v, acc_sh.at[pl.ds(s * stripe + t * 16, 16)])

    plsc.subcore_barrier()

    @pl.loop(0, KW)
    def _(j):
        pltpu.sync_copy(dstm_hbm.at[wid * KW + j], idx_v)
        pltpu.sync_copy(msg_hbm.at[pl.ds(wid * EPW + j * WIN, WIN)], upd_v)
        pltpu.sync_copy(upd_v, acc_sh.at[idx_v], add=True)

    plsc.subcore_barrier()
    pltpu.sync_copy(acc_sh.at[pl.ds(s * stripe, stripe)],
                    out_hbm.at[c, pl.ds(s * stripe, stripe)])


def _sc_scatter(msg, dstm):
    return pl.kernel(
        _scatter_body,
        out_type=jax.ShapeDtypeStruct((2, N_PAD, D_EXT), jnp.float32),
        mesh=_vec_mesh(),
        scratch_types=[
            pltpu.VMEM((WIN,), jnp.int32),
            pltpu.VMEM((WIN, D_EXT), jnp.float32),
            pltpu.VMEM((16, D_EXT), jnp.float32),
            pltpu.VMEM_SHARED((N_PAD, D_EXT), jnp.float32),
            pltpu.SemaphoreType.DMA,
        ],
        compiler_params=pltpu.CompilerParams(use_tc_tiling_on_sc=False),
    )(msg, dstm)


_BE = 2048


def _msg_body(ea_ref, xs_ref, W1_ref, b1_ref, W2b_ref, b2r_ref, out_ref):
    h = jnp.maximum(
        jnp.dot(ea_ref[...], W1_ref[...], preferred_element_type=jnp.float32)
        + b1_ref[...], 0.0)
    xs = xs_ref[...]
    xs8 = xs[:, :NODE_DIM]
    q = jnp.dot(h.astype(jnp.bfloat16), W2b_ref[...],
                preferred_element_type=jnp.float32)
    sel = (lax.broadcasted_iota(jnp.int32, (NODE_DIM, 256), 1) // 32
           == lax.broadcasted_iota(jnp.int32, (NODE_DIM, 256), 0)
           ).astype(jnp.bfloat16)
    xsw = jnp.dot(xs8.astype(jnp.bfloat16), sel,
                  preferred_element_type=jnp.float32)
    t = q * xsw
    r = t[:, :128] + t[:, 128:]
    m = r[:, :32] + r[:, 32:64] + r[:, 64:96] + r[:, 96:128]
    acc = m + jnp.dot(xs8, b2r_ref[...], preferred_element_type=jnp.float32)
    out_ref[...] = jnp.concatenate(
        [acc,
         jnp.ones((_BE, 1), jnp.float32),
         jnp.zeros((_BE, D_EXT - HIDDEN_DIM - 1), jnp.float32)], axis=1)


def _tc_messages(ea_p, xs, W1, b1r, W2b, b2r):
    grid = E_PAD // _BE
    return pl.pallas_call(
        _msg_body,
        grid=(grid,),
        in_specs=[
            pl.BlockSpec((_BE, EDGE_DIM), lambda i: (i, 0)),
            pl.BlockSpec((_BE, 16), lambda i: (i, 0)),
            pl.BlockSpec((EDGE_DIM, 32), lambda i: (0, 0)),
            pl.BlockSpec((1, 32), lambda i: (0, 0)),
            pl.BlockSpec((32, NODE_DIM * 32), lambda i: (0, 0)),
            pl.BlockSpec((NODE_DIM, HIDDEN_DIM), lambda i: (0, 0)),
        ],
        out_specs=pl.BlockSpec((_BE, D_EXT), lambda i: (i, 0)),
        out_shape=jax.ShapeDtypeStruct((E_PAD, D_EXT), jnp.float32),
        compiler_params=pltpu.CompilerParams(
            dimension_semantics=("parallel",)),
    )(ea_p, xs, W1, b1r, W2b, b2r)


_TN = 2000
_NSTEPS = N_NODES // _TN


def _finish_body(acc_ref, x_ref, batch_ref, Wr_ref, bc_ref, ids_ref, rat_ref,
                 emb_ref, W0a_ref, W0b_ref, b0_ref, Wf1_ref, bf1_ref,
                 Wf2_ref, bf2_ref, out_ref, psum_ref, pcnt_ref):
    i = pl.program_id(0)

    @pl.when(i == 0)
    def _():
        psum_ref[...] = jnp.zeros_like(psum_ref)
        pcnt_ref[...] = jnp.zeros_like(pcnt_ref)

    a0 = acc_ref[0]
    a1 = acc_ref[1]
    msum = a0[:, :HIDDEN_DIM] + a1[:, :HIDDEN_DIM]
    cnt = a0[:, HIDDEN_DIM:HIDDEN_DIM + 1] + a1[:, HIDDEN_DIM:HIDDEN_DIM + 1]
    agg = msum / jnp.maximum(cnt, 1.0)
    h1 = jnp.maximum(
        jnp.dot(x_ref[...], Wr_ref[...], preferred_element_type=jnp.float32)
        + agg + bc_ref[...], 0.0)
    oh = (lax.broadcasted_iota(jnp.int32, (N_GRAPHS, _TN), 0)
          == batch_ref[0]).astype(jnp.float32)
    psum_ref[...] += jnp.dot(oh, h1, preferred_element_type=jnp.float32)
    pcnt_ref[...] += jnp.dot(oh, jnp.ones((_TN, HIDDEN_DIM), jnp.float32),
                             preferred_element_type=jnp.float32)

    @pl.when(i == _NSTEPS - 1)
    def _():
        pooled = psum_ref[...] / jnp.maximum(pcnt_ref[...], 1.0)
        sel = (ids_ref[...] == lax.broadcasted_iota(jnp.int32, (8, VOCAB), 1)
               ).astype(jnp.float32) * rat_ref[...]
        u = jnp.sum(jnp.dot(sel, emb_ref[...],
                            preferred_element_type=jnp.float32),
                    axis=0, keepdims=True)
        z1 = jnp.maximum(
            jnp.dot(pooled, W0a_ref[...], preferred_element_type=jnp.float32)
            + jnp.dot(u, W0b_ref[...], preferred_element_type=jnp.float32)
            + b0_ref[...], 0.0)
        z2 = jnp.maximum(
            jnp.dot(z1, Wf1_ref[...], preferred_element_type=jnp.float32)
            + bf1_ref[...], 0.0)
        out_ref[...] = (jnp.dot(z2, Wf2_ref[...],
                                preferred_element_type=jnp.float32)
                        + bf2_ref[...])


def _tc_finish(accs, x, batch_row, W_root, bcr, ids_col, rat_col, emb_table,
               W0a, W0b, b0r, Wf1, bf1r, Wf2, bf2r):
    full = lambda a: pl.BlockSpec(a.shape, lambda i: tuple(0 for _ in a.shape))
    return pl.pallas_call(
        _finish_body,
        grid=(_NSTEPS,),
        in_specs=[
            pl.BlockSpec((2, _TN, D_EXT), lambda i: (0, i, 0)),
            pl.BlockSpec((_TN, NODE_DIM), lambda i: (i, 0)),
            pl.BlockSpec((1, 1, _TN), lambda i: (i, 0, 0)),
            full(W_root), full(bcr), full(ids_col), full(rat_col),
            full(emb_table), full(W0a), full(W0b), full(b0r),
            full(Wf1), full(bf1r), full(Wf2), full(bf2r),
        ],
        out_specs=pl.BlockSpec((N_GRAPHS, 1), lambda i: (0, 0)),
        out_shape=jax.ShapeDtypeStruct((N_GRAPHS, 1), jnp.float32),
        scratch_shapes=[
            pltpu.VMEM((N_GRAPHS, HIDDEN_DIM), jnp.float32),
            pltpu.VMEM((N_GRAPHS, HIDDEN_DIM), jnp.float32),
        ],
        compiler_params=pltpu.CompilerParams(
            dimension_semantics=("arbitrary",)),
    )(accs, x, batch_row, W_root, bcr, ids_col, rat_col, emb_table,
      W0a, W0b, b0r, Wf1, bf1r, Wf2, bf2r)


def kernel(x, edge_index, edge_attr, batch, the_ids, the_ratios,
           W1, b1, W2, b2, W_root, b_conv, emb_table, W0, b0,
           Wf1, bf1, Wf2, bf2):
    pad = E_PAD - N_EDGES
    src = edge_index[0]
    dst = edge_index[1]
    srcm = jnp.concatenate(
        [src, jnp.zeros((pad,), jnp.int32)]).reshape(E_PAD // WIN, WIN)
    dstm = jnp.concatenate(
        [dst, jnp.full((pad,), N_NODES, jnp.int32)]).reshape(E_PAD // WIN, WIN)
    ea_p = jnp.concatenate(
        [edge_attr, jnp.zeros((pad, EDGE_DIM), jnp.float32)], axis=0)
    xp = jnp.pad(x, ((0, 0), (0, 16 - NODE_DIM)))

    xs = _sc_gather(xp, srcm)
    msg = _tc_messages(ea_p, xs, W1, b1.reshape(1, 32),
                       W2.astype(jnp.bfloat16),
                       b2.reshape(NODE_DIM, HIDDEN_DIM))
    accs = _sc_scatter(msg, dstm)
    out = _tc_finish(accs, x, batch.reshape(_NSTEPS, 1, _TN), W_root,
                     b_conv.reshape(1, HIDDEN_DIM),
                     the_ids.reshape(8, 1), the_ratios.reshape(8, 1),
                     emb_table, W0[:HIDDEN_DIM], W0[HIDDEN_DIM:],
                     b0.reshape(1, 64), Wf1, bf1.reshape(1, 32),
                     Wf2, bf2.reshape(1, 1))
    return out


# speedup vs baseline: 1.4926x; 1.0843x over previous
import jax
import jax.numpy as jnp
from jax import lax
from jax.experimental import pallas as pl
from jax.experimental.pallas import tpu as pltpu
from jax.experimental.pallas import tpu_sc as plsc

N_NODES = 10000
N_EDGES = 320000
NODE_DIM = 8
EDGE_DIM = 4
HIDDEN_DIM = 32
EMBED_DIM = 64
N_GRAPHS = 64
VOCAB = 118

NW = 32
WIN = 128
KW = 80
EPW = WIN * KW
E_PAD = NW * EPW
N_PAD = 10240
D_EXT = 48

def _vec_mesh():
    return plsc.VectorSubcoreMesh(core_axis_name="c", subcore_axis_name="s",
                                  num_cores=2, num_subcores=16)


def _gather_body(xp_hbm, srcm_hbm, xs_hbm, idx_v, rows_v, sem):
    wid = lax.axis_index("c") * 16 + lax.axis_index("s")
    pltpu.sync_copy(srcm_hbm.at[pl.ds(wid * KW, KW)], idx_v)

    @pl.loop(0, KW)
    def _(j):
        pltpu.async_copy(xp_hbm.at[idx_v.at[j]], rows_v, sem).wait()
        pltpu.sync_copy(rows_v, xs_hbm.at[pl.ds(wid * EPW + j * WIN, WIN)])


def _sc_gather(xp, srcm):
    return pl.kernel(
        _gather_body,
        out_type=jax.ShapeDtypeStruct((E_PAD, 16), jnp.float32),
        mesh=_vec_mesh(),
        scratch_types=[
            pltpu.VMEM((KW, WIN), jnp.int32),
            pltpu.VMEM((WIN, 16), jnp.float32),
            pltpu.SemaphoreType.DMA,
        ],
        compiler_params=pltpu.CompilerParams(use_tc_tiling_on_sc=False),
    )(xp, srcm)


def _scatter_body(msg_hbm, dstm_hbm, out_hbm, idx_v, upd_v, zb_v, acc_sh, sem):
    c = lax.axis_index("c")
    s = lax.axis_index("s")
    wid = c * 16 + s
    stripe = N_PAD // 16

    @pl.loop(0, 16)
    def _(r):
        @pl.loop(0, D_EXT // 16)
        def _(cc):
            zb_v[r, pl.ds(cc * 16, 16)] = jnp.zeros((16,), jnp.float32)

    @pl.loop(0, stripe // 16)
    def _(t):
        pltpu.sync_copy(zb_v, acc_sh.at[pl.ds(s * stripe + t * 16, 16)])

    plsc.subcore_barrier()

    @pl.loop(0, KW)
    def _(j):
        pltpu.sync_copy(dstm_hbm.at[wid * KW + j], idx_v)
        pltpu.sync_copy(msg_hbm.at[pl.ds(wid * EPW + j * WIN, WIN)], upd_v)
        pltpu.sync_copy(upd_v, acc_sh.at[idx_v], add=True)

    plsc.subcore_barrier()
    pltpu.sync_copy(acc_sh.at[pl.ds(s * stripe, stripe)],
                    out_hbm.at[c, pl.ds(s * stripe, stripe)])


def _sc_scatter(msg, dstm):
    return pl.kernel(
        _scatter_body,
        out_type=jax.ShapeDtypeStruct((2, N_PAD, D_EXT), jnp.float32),
        mesh=_vec_mesh(),
        scratch_types=[
            pltpu.VMEM((WIN,), jnp.int32),
            pltpu.VMEM((WIN, D_EXT), jnp.float32),
            pltpu.VMEM((16, D_EXT), jnp.float32),
            pltpu.VMEM_SHARED((N_PAD, D_EXT), jnp.float32),
            pltpu.SemaphoreType.DMA,
        ],
        compiler_params=pltpu.CompilerParams(use_tc_tiling_on_sc=False),
    )(msg, dstm)


_BE = 8192


def _msg_body(ea_ref, xs_ref, W1_ref, b1_ref, W2b_ref, b2r_ref, out_ref):
    h = jnp.maximum(
        jnp.dot(ea_ref[...], W1_ref[...], preferred_element_type=jnp.float32)
        + b1_ref[...], 0.0)
    xs = xs_ref[...]
    xs8 = xs[:, :NODE_DIM]
    q = jnp.dot(h.astype(jnp.bfloat16), W2b_ref[...],
                preferred_element_type=jnp.float32)
    sel = (lax.broadcasted_iota(jnp.int32, (NODE_DIM, 256), 1) // 32
           == lax.broadcasted_iota(jnp.int32, (NODE_DIM, 256), 0)
           ).astype(jnp.bfloat16)
    xsw = jnp.dot(xs8.astype(jnp.bfloat16), sel,
                  preferred_element_type=jnp.float32)
    t = q * xsw
    r = t[:, :128] + t[:, 128:]
    m = r[:, :32] + r[:, 32:64] + r[:, 64:96] + r[:, 96:128]
    acc = m + jnp.dot(xs8, b2r_ref[...], preferred_element_type=jnp.float32)
    out_ref[...] = jnp.concatenate(
        [acc,
         jnp.ones((_BE, 1), jnp.float32),
         jnp.zeros((_BE, D_EXT - HIDDEN_DIM - 1), jnp.float32)], axis=1)


def _tc_messages(ea_p, xs, W1, b1r, W2b, b2r):
    grid = E_PAD // _BE
    return pl.pallas_call(
        _msg_body,
        grid=(grid,),
        in_specs=[
            pl.BlockSpec((_BE, EDGE_DIM), lambda i: (i, 0)),
            pl.BlockSpec((_BE, 16), lambda i: (i, 0)),
            pl.BlockSpec((EDGE_DIM, 32), lambda i: (0, 0)),
            pl.BlockSpec((1, 32), lambda i: (0, 0)),
            pl.BlockSpec((32, NODE_DIM * 32), lambda i: (0, 0)),
            pl.BlockSpec((NODE_DIM, HIDDEN_DIM), lambda i: (0, 0)),
        ],
        out_specs=pl.BlockSpec((_BE, D_EXT), lambda i: (i, 0)),
        out_shape=jax.ShapeDtypeStruct((E_PAD, D_EXT), jnp.float32),
        compiler_params=pltpu.CompilerParams(
            dimension_semantics=("parallel",)),
    )(ea_p, xs, W1, b1r, W2b, b2r)


_TN = 2000
_NSTEPS = N_NODES // _TN


def _finish_body(acc_ref, x_ref, batch_ref, Wr_ref, bc_ref, ids_ref, rat_ref,
                 emb_ref, W0a_ref, W0b_ref, b0_ref, Wf1_ref, bf1_ref,
                 Wf2_ref, bf2_ref, out_ref, psum_ref, pcnt_ref):
    i = pl.program_id(0)

    @pl.when(i == 0)
    def _():
        psum_ref[...] = jnp.zeros_like(psum_ref)
        pcnt_ref[...] = jnp.zeros_like(pcnt_ref)

    a0 = acc_ref[0]
    a1 = acc_ref[1]
    msum = a0[:, :HIDDEN_DIM] + a1[:, :HIDDEN_DIM]
    cnt = a0[:, HIDDEN_DIM:HIDDEN_DIM + 1] + a1[:, HIDDEN_DIM:HIDDEN_DIM + 1]
    agg = msum / jnp.maximum(cnt, 1.0)
    h1 = jnp.maximum(
        jnp.dot(x_ref[...], Wr_ref[...], preferred_element_type=jnp.float32)
        + agg + bc_ref[...], 0.0)
    oh = (lax.broadcasted_iota(jnp.int32, (N_GRAPHS, _TN), 0)
          == batch_ref[0]).astype(jnp.float32)
    psum_ref[...] += jnp.dot(oh, h1, preferred_element_type=jnp.float32)
    pcnt_ref[...] += jnp.dot(oh, jnp.ones((_TN, HIDDEN_DIM), jnp.float32),
                             preferred_element_type=jnp.float32)

    @pl.when(i == _NSTEPS - 1)
    def _():
        pooled = psum_ref[...] / jnp.maximum(pcnt_ref[...], 1.0)
        sel = (ids_ref[...] == lax.broadcasted_iota(jnp.int32, (8, VOCAB), 1)
               ).astype(jnp.float32) * rat_ref[...]
        u = jnp.sum(jnp.dot(sel, emb_ref[...],
                            preferred_element_type=jnp.float32),
                    axis=0, keepdims=True)
        z1 = jnp.maximum(
            jnp.dot(pooled, W0a_ref[...], preferred_element_type=jnp.float32)
            + jnp.dot(u, W0b_ref[...], preferred_element_type=jnp.float32)
            + b0_ref[...], 0.0)
        z2 = jnp.maximum(
            jnp.dot(z1, Wf1_ref[...], preferred_element_type=jnp.float32)
            + bf1_ref[...], 0.0)
        out_ref[...] = (jnp.dot(z2, Wf2_ref[...],
                                preferred_element_type=jnp.float32)
                        + bf2_ref[...])


def _tc_finish(accs, x, batch_row, W_root, bcr, ids_col, rat_col, emb_table,
               W0a, W0b, b0r, Wf1, bf1r, Wf2, bf2r):
    full = lambda a: pl.BlockSpec(a.shape, lambda i: tuple(0 for _ in a.shape))
    return pl.pallas_call(
        _finish_body,
        grid=(_NSTEPS,),
        in_specs=[
            pl.BlockSpec((2, _TN, D_EXT), lambda i: (0, i, 0)),
            pl.BlockSpec((_TN, NODE_DIM), lambda i: (i, 0)),
            pl.BlockSpec((1, 1, _TN), lambda i: (i, 0, 0)),
            full(W_root), full(bcr), full(ids_col), full(rat_col),
            full(emb_table), full(W0a), full(W0b), full(b0r),
            full(Wf1), full(bf1r), full(Wf2), full(bf2r),
        ],
        out_specs=pl.BlockSpec((N_GRAPHS, 1), lambda i: (0, 0)),
        out_shape=jax.ShapeDtypeStruct((N_GRAPHS, 1), jnp.float32),
        scratch_shapes=[
            pltpu.VMEM((N_GRAPHS, HIDDEN_DIM), jnp.float32),
            pltpu.VMEM((N_GRAPHS, HIDDEN_DIM), jnp.float32),
        ],
        compiler_params=pltpu.CompilerParams(
            dimension_semantics=("arbitrary",)),
    )(accs, x, batch_row, W_root, bcr, ids_col, rat_col, emb_table,
      W0a, W0b, b0r, Wf1, bf1r, Wf2, bf2r)


def kernel(x, edge_index, edge_attr, batch, the_ids, the_ratios,
           W1, b1, W2, b2, W_root, b_conv, emb_table, W0, b0,
           Wf1, bf1, Wf2, bf2):
    pad = E_PAD - N_EDGES
    src = edge_index[0]
    dst = edge_index[1]
    srcm = jnp.concatenate(
        [src, jnp.zeros((pad,), jnp.int32)]).reshape(E_PAD // WIN, WIN)
    dstm = jnp.concatenate(
        [dst, jnp.full((pad,), N_NODES, jnp.int32)]).reshape(E_PAD // WIN, WIN)
    ea_p = jnp.concatenate(
        [edge_attr, jnp.zeros((pad, EDGE_DIM), jnp.float32)], axis=0)
    xp = jnp.pad(x, ((0, 0), (0, 16 - NODE_DIM)))

    xs = _sc_gather(xp, srcm)
    msg = _tc_messages(ea_p, xs, W1, b1.reshape(1, 32),
                       W2.astype(jnp.bfloat16),
                       b2.reshape(NODE_DIM, HIDDEN_DIM))
    accs = _sc_scatter(msg, dstm)
    out = _tc_finish(accs, x, batch.reshape(_NSTEPS, 1, _TN), W_root,
                     b_conv.reshape(1, HIDDEN_DIM),
                     the_ids.reshape(8, 1), the_ratios.reshape(8, 1),
                     emb_table, W0[:HIDDEN_DIM], W0[HIDDEN_DIM:],
                     b0.reshape(1, 64), Wf1, bf1.reshape(1, 32),
                     Wf2, bf2.reshape(1, 1))
    return out


# speedup vs baseline: 1.6417x; 1.0999x over previous
import jax
import jax.numpy as jnp
from jax import lax
from jax.experimental import pallas as pl
from jax.experimental.pallas import tpu as pltpu
from jax.experimental.pallas import tpu_sc as plsc

N_NODES = 10000
N_EDGES = 320000
NODE_DIM = 8
EDGE_DIM = 4
HIDDEN_DIM = 32
EMBED_DIM = 64
N_GRAPHS = 64
VOCAB = 118

NW = 32
WIN = 128
KW = 80
EPW = WIN * KW
E_PAD = NW * EPW
N_PAD = 10240
D_EXT = 48

def _vec_mesh():
    return plsc.VectorSubcoreMesh(core_axis_name="c", subcore_axis_name="s",
                                  num_cores=2, num_subcores=16)


_GDEPTH = 4


def _gather_body(xp_hbm, srcm_hbm, xs_hbm, idx_v, rows_v, gsems, wsems):
    wid = lax.axis_index("c") * 16 + lax.axis_index("s")
    pltpu.sync_copy(srcm_hbm.at[pl.ds(wid * KW, KW)], idx_v)

    @pl.loop(0, KW, step=_GDEPTH)
    def _(j):
        gs = [pltpu.async_copy(xp_hbm.at[idx_v.at[j + b]], rows_v.at[b],
                               gsems.at[b]) for b in range(_GDEPTH)]
        ws = []
        for b in range(_GDEPTH):
            gs[b].wait()
            ws.append(pltpu.async_copy(
                rows_v.at[b],
                xs_hbm.at[pl.ds(wid * EPW + (j + b) * WIN, WIN)],
                wsems.at[b]))
        for b in range(_GDEPTH):
            ws[b].wait()


def _sc_gather(xp, srcm):
    return pl.kernel(
        _gather_body,
        out_type=jax.ShapeDtypeStruct((E_PAD, 16), jnp.float32),
        mesh=_vec_mesh(),
        scratch_types=[
            pltpu.VMEM((KW, WIN), jnp.int32),
            pltpu.VMEM((_GDEPTH, WIN, 16), jnp.float32),
            pltpu.SemaphoreType.DMA((_GDEPTH,)),
            pltpu.SemaphoreType.DMA((_GDEPTH,)),
        ],
        compiler_params=pltpu.CompilerParams(use_tc_tiling_on_sc=False),
    )(xp, srcm)


def _scatter_body(msg_hbm, dstm_hbm, out_hbm, idx_v, idx_v1, upd_v, upd_v1,
                  zb_v, acc_sh, sem, isem0, isem1, usem0, usem1):
    c = lax.axis_index("c")
    s = lax.axis_index("s")
    wid = c * 16 + s
    stripe = N_PAD // 16

    @pl.loop(0, 16)
    def _(r):
        @pl.loop(0, D_EXT // 16)
        def _(cc):
            zb_v[r, pl.ds(cc * 16, 16)] = jnp.zeros((16,), jnp.float32)

    @pl.loop(0, stripe // 16)
    def _(t):
        pltpu.sync_copy(zb_v, acc_sh.at[pl.ds(s * stripe + t * 16, 16)])

    plsc.subcore_barrier()

    @pl.loop(0, KW, step=2)
    def _(j):
        l0 = pltpu.async_copy(dstm_hbm.at[wid * KW + j], idx_v, isem0)
        u0 = pltpu.async_copy(msg_hbm.at[pl.ds(wid * EPW + j * WIN, WIN)],
                              upd_v, usem0)
        l1 = pltpu.async_copy(dstm_hbm.at[wid * KW + j + 1], idx_v1, isem1)
        u1 = pltpu.async_copy(
            msg_hbm.at[pl.ds(wid * EPW + (j + 1) * WIN, WIN)], upd_v1, usem1)
        l0.wait()
        u0.wait()
        pltpu.sync_copy(upd_v, acc_sh.at[idx_v], add=True)
        l1.wait()
        u1.wait()
        pltpu.sync_copy(upd_v1, acc_sh.at[idx_v1], add=True)

    plsc.subcore_barrier()
    pltpu.sync_copy(acc_sh.at[pl.ds(s * stripe, stripe)],
                    out_hbm.at[c, pl.ds(s * stripe, stripe)])


def _sc_scatter(msg, dstm):
    return pl.kernel(
        _scatter_body,
        out_type=jax.ShapeDtypeStruct((2, N_PAD, D_EXT), jnp.float32),
        mesh=_vec_mesh(),
        scratch_types=[
            pltpu.VMEM((WIN,), jnp.int32),
            pltpu.VMEM((WIN,), jnp.int32),
            pltpu.VMEM((WIN, D_EXT), jnp.float32),
            pltpu.VMEM((WIN, D_EXT), jnp.float32),
            pltpu.VMEM((16, D_EXT), jnp.float32),
            pltpu.VMEM_SHARED((N_PAD, D_EXT), jnp.float32),
            pltpu.SemaphoreType.DMA,
            pltpu.SemaphoreType.DMA,
            pltpu.SemaphoreType.DMA,
            pltpu.SemaphoreType.DMA,
            pltpu.SemaphoreType.DMA,
        ],
        compiler_params=pltpu.CompilerParams(use_tc_tiling_on_sc=False),
    )(msg, dstm)


_BE = 8192


def _msg_body(ea_ref, xs_ref, W1_ref, b1_ref, W2b_ref, b2r_ref, out_ref):
    h = jnp.maximum(
        jnp.dot(ea_ref[...], W1_ref[...], preferred_element_type=jnp.float32)
        + b1_ref[...], 0.0)
    xs = xs_ref[...]
    xs8 = xs[:, :NODE_DIM]
    q = jnp.dot(h.astype(jnp.bfloat16), W2b_ref[...],
                preferred_element_type=jnp.float32)
    sel = (lax.broadcasted_iota(jnp.int32, (NODE_DIM, 256), 1) // 32
           == lax.broadcasted_iota(jnp.int32, (NODE_DIM, 256), 0)
           ).astype(jnp.bfloat16)
    xsw = jnp.dot(xs8.astype(jnp.bfloat16), sel,
                  preferred_element_type=jnp.float32)
    t = q * xsw
    r = t[:, :128] + t[:, 128:]
    m = r[:, :32] + r[:, 32:64] + r[:, 64:96] + r[:, 96:128]
    acc = m + jnp.dot(xs8, b2r_ref[...], preferred_element_type=jnp.float32)
    out_ref[...] = jnp.concatenate(
        [acc,
         jnp.ones((_BE, 1), jnp.float32),
         jnp.zeros((_BE, D_EXT - HIDDEN_DIM - 1), jnp.float32)], axis=1)


def _tc_messages(ea_p, xs, W1, b1r, W2b, b2r):
    grid = E_PAD // _BE
    return pl.pallas_call(
        _msg_body,
        grid=(grid,),
        in_specs=[
            pl.BlockSpec((_BE, EDGE_DIM), lambda i: (i, 0)),
            pl.BlockSpec((_BE, 16), lambda i: (i, 0)),
            pl.BlockSpec((EDGE_DIM, 32), lambda i: (0, 0)),
            pl.BlockSpec((1, 32), lambda i: (0, 0)),
            pl.BlockSpec((32, NODE_DIM * 32), lambda i: (0, 0)),
            pl.BlockSpec((NODE_DIM, HIDDEN_DIM), lambda i: (0, 0)),
        ],
        out_specs=pl.BlockSpec((_BE, D_EXT), lambda i: (i, 0)),
        out_shape=jax.ShapeDtypeStruct((E_PAD, D_EXT), jnp.float32),
        compiler_params=pltpu.CompilerParams(
            dimension_semantics=("parallel",)),
    )(ea_p, xs, W1, b1r, W2b, b2r)


_TN = 2000
_NSTEPS = N_NODES // _TN


def _finish_body(acc_ref, x_ref, batch_ref, Wr_ref, bc_ref, ids_ref, rat_ref,
                 emb_ref, W0a_ref, W0b_ref, b0_ref, Wf1_ref, bf1_ref,
                 Wf2_ref, bf2_ref, out_ref, psum_ref, pcnt_ref):
    i = pl.program_id(0)

    @pl.when(i == 0)
    def _():
        psum_ref[...] = jnp.zeros_like(psum_ref)
        pcnt_ref[...] = jnp.zeros_like(pcnt_ref)

    a0 = acc_ref[0]
    a1 = acc_ref[1]
    msum = a0[:, :HIDDEN_DIM] + a1[:, :HIDDEN_DIM]
    cnt = a0[:, HIDDEN_DIM:HIDDEN_DIM + 1] + a1[:, HIDDEN_DIM:HIDDEN_DIM + 1]
    agg = msum / jnp.maximum(cnt, 1.0)
    h1 = jnp.maximum(
        jnp.dot(x_ref[...], Wr_ref[...], preferred_element_type=jnp.float32)
        + agg + bc_ref[...], 0.0)
    oh = (lax.broadcasted_iota(jnp.int32, (N_GRAPHS, _TN), 0)
          == batch_ref[0]).astype(jnp.float32)
    psum_ref[...] += jnp.dot(oh, h1, preferred_element_type=jnp.float32)
    pcnt_ref[...] += jnp.dot(oh, jnp.ones((_TN, HIDDEN_DIM), jnp.float32),
                             preferred_element_type=jnp.float32)

    @pl.when(i == _NSTEPS - 1)
    def _():
        pooled = psum_ref[...] / jnp.maximum(pcnt_ref[...], 1.0)
        sel = (ids_ref[...] == lax.broadcasted_iota(jnp.int32, (8, VOCAB), 1)
               ).astype(jnp.float32) * rat_ref[...]
        u = jnp.sum(jnp.dot(sel, emb_ref[...],
                            preferred_element_type=jnp.float32),
                    axis=0, keepdims=True)
        z1 = jnp.maximum(
            jnp.dot(pooled, W0a_ref[...], preferred_element_type=jnp.float32)
            + jnp.dot(u, W0b_ref[...], preferred_element_type=jnp.float32)
            + b0_ref[...], 0.0)
        z2 = jnp.maximum(
            jnp.dot(z1, Wf1_ref[...], preferred_element_type=jnp.float32)
            + bf1_ref[...], 0.0)
        out_ref[...] = (jnp.dot(z2, Wf2_ref[...],
                                preferred_element_type=jnp.float32)
                        + bf2_ref[...])


def _tc_finish(accs, x, batch_row, W_root, bcr, ids_col, rat_col, emb_table,
               W0a, W0b, b0r, Wf1, bf1r, Wf2, bf2r):
    full = lambda a: pl.BlockSpec(a.shape, lambda i: tuple(0 for _ in a.shape))
    return pl.pallas_call(
        _finish_body,
        grid=(_NSTEPS,),
        in_specs=[
            pl.BlockSpec((2, _TN, D_EXT), lambda i: (0, i, 0)),
            pl.BlockSpec((_TN, NODE_DIM), lambda i: (i, 0)),
            pl.BlockSpec((1, 1, _TN), lambda i: (i, 0, 0)),
            full(W_root), full(bcr), full(ids_col), full(rat_col),
            full(emb_table), full(W0a), full(W0b), full(b0r),
            full(Wf1), full(bf1r), full(Wf2), full(bf2r),
        ],
        out_specs=pl.BlockSpec((N_GRAPHS, 1), lambda i: (0, 0)),
        out_shape=jax.ShapeDtypeStruct((N_GRAPHS, 1), jnp.float32),
        scratch_shapes=[
            pltpu.VMEM((N_GRAPHS, HIDDEN_DIM), jnp.float32),
            pltpu.VMEM((N_GRAPHS, HIDDEN_DIM), jnp.float32),
        ],
        compiler_params=pltpu.CompilerParams(
            dimension_semantics=("arbitrary",)),
    )(accs, x, batch_row, W_root, bcr, ids_col, rat_col, emb_table,
      W0a, W0b, b0r, Wf1, bf1r, Wf2, bf2r)


def kernel(x, edge_index, edge_attr, batch, the_ids, the_ratios,
           W1, b1, W2, b2, W_root, b_conv, emb_table, W0, b0,
           Wf1, bf1, Wf2, bf2):
    pad = E_PAD - N_EDGES
    src = edge_index[0]
    dst = edge_index[1]
    srcm = jnp.concatenate(
        [src, jnp.zeros((pad,), jnp.int32)]).reshape(E_PAD // WIN, WIN)
    dstm = jnp.concatenate(
        [dst, jnp.full((pad,), N_NODES, jnp.int32)]).reshape(E_PAD // WIN, WIN)
    ea_p = jnp.concatenate(
        [edge_attr, jnp.zeros((pad, EDGE_DIM), jnp.float32)], axis=0)
    xp = jnp.pad(x, ((0, 0), (0, 16 - NODE_DIM)))

    xs = _sc_gather(xp, srcm)
    msg = _tc_messages(ea_p, xs, W1, b1.reshape(1, 32),
                       W2.astype(jnp.bfloat16),
                       b2.reshape(NODE_DIM, HIDDEN_DIM))
    accs = _sc_scatter(msg, dstm)
    out = _tc_finish(accs, x, batch.reshape(_NSTEPS, 1, _TN), W_root,
                     b_conv.reshape(1, HIDDEN_DIM),
                     the_ids.reshape(8, 1), the_ratios.reshape(8, 1),
                     emb_table, W0[:HIDDEN_DIM], W0[HIDDEN_DIM:],
                     b0.reshape(1, 64), Wf1, bf1.reshape(1, 32),
                     Wf2, bf2.reshape(1, 1))
    return out
